# baseline (speedup 1.0000x reference)
import functools

import jax
import jax.numpy as jnp
from jax import lax
from jax.experimental import pallas as pl
from jax.experimental.pallas import tpu as pltpu
from jax.experimental.pallas import tpu_sc as plsc

NC = 2
NS = 16
L = 16
CH = 80
NB = 4
LAG = 2
SEP = 4000
FB = 80


def _body(xs_hbm, e0_hbm, e1_hbm, cnt_hbm, out_hbm,
          idx0all, idx1all, idxs0, idxs1, idxg0, idxg1, rows0, rows1,
          fbuf, cntb, acc, sg0, sg1, ss0, ss1):
  n_nodes = cnt_hbm.shape[0]
  n_edges = e0_hbm.shape[0]
  ep = n_edges // NS
  nseg = ep // SEP
  ncs = SEP // CH
  nj = n_nodes // FB
  nft = (nj + NS - 1) // NS

  c = lax.axis_index("c")
  s = lax.axis_index("s")

  zero16 = jnp.zeros((L,), jnp.float32)

  def zrow(i, carry):
    for k in range(4):
      fbuf[i, pl.ds(k * L, L)] = zero16
    return carry
  lax.fori_loop(0, FB, zrow, 0)
  for t in range(nft):
    j = s + NS * t

    @pl.when(j < nj)
    def _():
      pltpu.sync_copy(fbuf, acc.at[pl.ds(j * FB, FB)])

  plsc.subcore_barrier()

  def wait_scatters(b):
    pltpu.make_async_copy(rows0[b], acc.at[idxs0[b]], ss0[b]).wait()
    pltpu.make_async_copy(rows0[b], acc.at[idxs1[b]], ss1[b]).wait()

  def prep(b, jc):
    o = jc * CH
    for k in range(CH // L):
      sl = pl.ds(k * L, L)
      v0 = idx0all[pl.ds(o + k * L, L)]
      v1 = idx1all[pl.ds(o + k * L, L)]
      idxs0[b][sl] = v0
      idxs1[b][sl] = v1
      idxg0[b][sl] = 2 * v0 + c
      idxg1[b][sl] = 2 * v1 + c
    pltpu.make_async_copy(xs_hbm.at[idxg0[b]], rows0[b], sg0[b]).start()
    pltpu.make_async_copy(xs_hbm.at[idxg1[b]], rows1[b], sg1[b]).start()

  def finish(b):
    pltpu.make_async_copy(xs_hbm.at[idxg0[b]], rows0[b], sg0[b]).wait()
    pltpu.make_async_copy(xs_hbm.at[idxg1[b]], rows1[b], sg1[b]).wait()

    def add_row(i, inner):
      i4 = i * 4
      for r in range(4):
        for k in range(4):
          sl = pl.ds(k * L, L)
          rows0[b][i4 + r, sl] = rows0[b][i4 + r, sl] + rows1[b][i4 + r, sl]
      return inner
    lax.fori_loop(0, CH // 4, add_row, 0)

    pltpu.make_async_copy(rows0[b], acc.at[idxs0[b]], ss0[b]).start(add=True)
    pltpu.make_async_copy(rows0[b], acc.at[idxs1[b]], ss1[b]).start(add=True)

  def seg_body(seg, carry):
    base = s * ep + seg * SEP
    pltpu.sync_copy(e0_hbm.at[pl.ds(base, SEP)], idx0all)
    pltpu.sync_copy(e1_hbm.at[pl.ds(base, SEP)], idx1all)
    for b in range(LAG):
      @pl.when(seg > 0)
      def _():
        wait_scatters(b)
      prep(b, b)

    def ring(jj, carry2):
      for b in range(NB):
        jc = NB * jj + b

        @pl.when(jc < ncs)
        def _():
          finish(b)
        pjc = jc + LAG
        pb = (b + LAG) % NB

        @pl.when(pjc < ncs)
        def _():
          @pl.when(jnp.logical_or(seg > 0, pjc >= NB))
          def _():
            wait_scatters(pb)
          prep(pb, pjc)
      return carry2
    lax.fori_loop(0, (ncs + NB - 1) // NB, ring, 0)
    return carry
  lax.fori_loop(0, nseg, seg_body, 0)
  for b in range(NB):
    wait_scatters(b)

  plsc.subcore_barrier()

  for t in range(nft):
    j = s + NS * t

    @pl.when(j < nj)
    def _():
      r0 = j * FB
      pltpu.sync_copy(acc.at[pl.ds(r0, FB)], fbuf)
      pltpu.sync_copy(cnt_hbm.at[pl.ds(r0, FB)], cntb.at[pl.ds(0, FB)])

      def fin_row(i, carry):
        cv = cntb[pl.ds(i, L)]
        scale = (0.5 / cv)[0]
        for k in range(4):
          sl = pl.ds(k * L, L)
          fbuf[i, sl] = fbuf[i, sl] * scale
        return carry
      lax.fori_loop(0, FB, fin_row, 0)
      pltpu.sync_copy(fbuf, out_hbm.at[pl.ds(c * n_nodes + r0, FB)])


@jax.jit
def kernel(to_concat, node_features, edges, count):
  n_nodes, d = node_features.shape
  dh = d // 2
  e = edges.astype(jnp.int32)
  e0 = e[:, 0]
  e1 = e[:, 1]
  xs = node_features.reshape(2 * n_nodes, dh)
  cnt = count.reshape(n_nodes)

  mesh = plsc.VectorSubcoreMesh(
      core_axis_name="c", subcore_axis_name="s",
      num_cores=NC, num_subcores=NS)
  smooth = pl.kernel(
      _body,
      out_type=jax.ShapeDtypeStruct((2 * n_nodes, dh), jnp.float32),
      mesh=mesh,
      compiler_params=pltpu.CompilerParams(use_tc_tiling_on_sc=False),
      scratch_types=[
          pltpu.VMEM((SEP,), jnp.int32),
          pltpu.VMEM((SEP,), jnp.int32),
          [pltpu.VMEM((CH,), jnp.int32)] * NB,
          [pltpu.VMEM((CH,), jnp.int32)] * NB,
          [pltpu.VMEM((CH,), jnp.int32)] * NB,
          [pltpu.VMEM((CH,), jnp.int32)] * NB,
          [pltpu.VMEM((CH, dh), jnp.float32)] * NB,
          [pltpu.VMEM((CH, dh), jnp.float32)] * NB,
          pltpu.VMEM((FB, dh), jnp.float32),
          pltpu.VMEM((FB + L,), jnp.float32),
          pltpu.VMEM_SHARED((n_nodes, dh), jnp.float32),
          [pltpu.SemaphoreType.DMA] * NB,
          [pltpu.SemaphoreType.DMA] * NB,
          [pltpu.SemaphoreType.DMA] * NB,
          [pltpu.SemaphoreType.DMA] * NB,
      ],
  )
  outs = smooth(xs, e0, e1, cnt)
  smoothed = outs.reshape(2, n_nodes, dh).transpose(1, 0, 2).reshape(n_nodes, d)
  return jnp.concatenate([to_concat, smoothed], axis=1)


# speedup vs baseline: 1.0452x; 1.0452x over previous
import functools

import jax
import jax.numpy as jnp
from jax import lax
from jax.experimental import pallas as pl
from jax.experimental.pallas import tpu as pltpu
from jax.experimental.pallas import tpu_sc as plsc

NC = 2
NS = 16
L = 16
CH = 80
NB = 6
LAG = 4
SEP = 4000
FB = 80


def _body(xs_hbm, e0_hbm, e1_hbm, cnt_hbm, out_hbm,
          idx0all, idx1all, idxs0, idxs1, idxg0, idxg1, rows0, rows1,
          fbuf, cntb, acc, sg0, sg1, ss0, ss1):
  n_nodes = cnt_hbm.shape[0]
  n_edges = e0_hbm.shape[0]
  ep = n_edges // NS
  nseg = ep // SEP
  ncs = SEP // CH
  nj = n_nodes // FB
  nft = (nj + NS - 1) // NS

  c = lax.axis_index("c")
  s = lax.axis_index("s")

  zero16 = jnp.zeros((L,), jnp.float32)

  def zrow(i, carry):
    for k in range(4):
      fbuf[i, pl.ds(k * L, L)] = zero16
    return carry
  lax.fori_loop(0, FB, zrow, 0)
  for t in range(nft):
    j = s + NS * t

    @pl.when(j < nj)
    def _():
      pltpu.sync_copy(fbuf, acc.at[pl.ds(j * FB, FB)])

  plsc.subcore_barrier()

  def wait_scatters(b):
    pltpu.make_async_copy(rows0[b], acc.at[idxs0[b]], ss0[b]).wait()
    pltpu.make_async_copy(rows0[b], acc.at[idxs1[b]], ss1[b]).wait()

  def prep(b, jc):
    o = jc * CH
    for k in range(CH // L):
      sl = pl.ds(k * L, L)
      v0 = idx0all[pl.ds(o + k * L, L)]
      v1 = idx1all[pl.ds(o + k * L, L)]
      idxs0[b][sl] = v0
      idxs1[b][sl] = v1
      idxg0[b][sl] = 2 * v0 + c
      idxg1[b][sl] = 2 * v1 + c
    pltpu.make_async_copy(xs_hbm.at[idxg0[b]], rows0[b], sg0[b]).start()
    pltpu.make_async_copy(xs_hbm.at[idxg1[b]], rows1[b], sg1[b]).start()

  def finish(b):
    pltpu.make_async_copy(xs_hbm.at[idxg0[b]], rows0[b], sg0[b]).wait()
    pltpu.make_async_copy(xs_hbm.at[idxg1[b]], rows1[b], sg1[b]).wait()

    def add_row(i, inner):
      i4 = i * 4
      for r in range(4):
        for k in range(4):
          sl = pl.ds(k * L, L)
          rows0[b][i4 + r, sl] = rows0[b][i4 + r, sl] + rows1[b][i4 + r, sl]
      return inner
    lax.fori_loop(0, CH // 4, add_row, 0)

    pltpu.make_async_copy(rows0[b], acc.at[idxs0[b]], ss0[b]).start(add=True)
    pltpu.make_async_copy(rows0[b], acc.at[idxs1[b]], ss1[b]).start(add=True)

  def seg_body(seg, carry):
    base = s * ep + seg * SEP
    pltpu.sync_copy(e0_hbm.at[pl.ds(base, SEP)], idx0all)
    pltpu.sync_copy(e1_hbm.at[pl.ds(base, SEP)], idx1all)
    for b in range(LAG):
      @pl.when(seg > 0)
      def _():
        wait_scatters(b)
      prep(b, b)

    def ring(jj, carry2):
      for b in range(NB):
        jc = NB * jj + b

        @pl.when(jc < ncs)
        def _():
          finish(b)
        pjc = jc + LAG
        pb = (b + LAG) % NB

        @pl.when(pjc < ncs)
        def _():
          @pl.when(jnp.logical_or(seg > 0, pjc >= NB))
          def _():
            wait_scatters(pb)
          prep(pb, pjc)
      return carry2
    lax.fori_loop(0, (ncs + NB - 1) // NB, ring, 0)
    return carry
  lax.fori_loop(0, nseg, seg_body, 0)
  for b in range(NB):
    wait_scatters(b)

  plsc.subcore_barrier()

  for t in range(nft):
    j = s + NS * t

    @pl.when(j < nj)
    def _():
      r0 = j * FB
      pltpu.sync_copy(acc.at[pl.ds(r0, FB)], fbuf)
      pltpu.sync_copy(cnt_hbm.at[pl.ds(r0, FB)], cntb.at[pl.ds(0, FB)])

      def fin_row(i, carry):
        cv = cntb[pl.ds(i, L)]
        scale = (0.5 / cv)[0]
        for k in range(4):
          sl = pl.ds(k * L, L)
          fbuf[i, sl] = fbuf[i, sl] * scale
        return carry
      lax.fori_loop(0, FB, fin_row, 0)
      pltpu.sync_copy(fbuf, out_hbm.at[pl.ds(c * n_nodes + r0, FB)])


@jax.jit
def kernel(to_concat, node_features, edges, count):
  n_nodes, d = node_features.shape
  dh = d // 2
  e = edges.astype(jnp.int32)
  e0 = e[:, 0]
  e1 = e[:, 1]
  xs = node_features.reshape(2 * n_nodes, dh)
  cnt = count.reshape(n_nodes)

  mesh = plsc.VectorSubcoreMesh(
      core_axis_name="c", subcore_axis_name="s",
      num_cores=NC, num_subcores=NS)
  smooth = pl.kernel(
      _body,
      out_type=jax.ShapeDtypeStruct((2 * n_nodes, dh), jnp.float32),
      mesh=mesh,
      compiler_params=pltpu.CompilerParams(use_tc_tiling_on_sc=False),
      scratch_types=[
          pltpu.VMEM((SEP,), jnp.int32),
          pltpu.VMEM((SEP,), jnp.int32),
          [pltpu.VMEM((CH,), jnp.int32)] * NB,
          [pltpu.VMEM((CH,), jnp.int32)] * NB,
          [pltpu.VMEM((CH,), jnp.int32)] * NB,
          [pltpu.VMEM((CH,), jnp.int32)] * NB,
          [pltpu.VMEM((CH, dh), jnp.float32)] * NB,
          [pltpu.VMEM((CH, dh), jnp.float32)] * NB,
          pltpu.VMEM((FB, dh), jnp.float32),
          pltpu.VMEM((FB + L,), jnp.float32),
          pltpu.VMEM_SHARED((n_nodes, dh), jnp.float32),
          [pltpu.SemaphoreType.DMA] * NB,
          [pltpu.SemaphoreType.DMA] * NB,
          [pltpu.SemaphoreType.DMA] * NB,
          [pltpu.SemaphoreType.DMA] * NB,
      ],
  )
  outs = smooth(xs, e0, e1, cnt)
  smoothed = outs.reshape(2, n_nodes, dh).transpose(1, 0, 2).reshape(n_nodes, d)
  return jnp.concatenate([to_concat, smoothed], axis=1)
